# baseline (speedup 1.0000x reference)
import functools

import jax
import jax.numpy as jnp
import numpy as np
from jax import lax
from jax.experimental import pallas as pl
from jax.experimental.pallas import tpu as pltpu
from jax.experimental.pallas import tpu_sc as plsc
from jax.experimental import layout as jex_layout

B, C, A, D = 1024, 50, 4, 64
VOCAB0, VOCAB1 = 100000, 1000
CA = C * A

NC, NS = 2, 16
NW = NC * NS
BPW = B // NW
NB = 2
NCHUNK = BPW // NB
GL = 56
EV = NB * GL


def _pos_encoding_np():
    pos = np.arange(C)[:, np.newaxis].astype(np.float32)
    i = np.arange(D)[np.newaxis, :].astype(np.float32)
    angle = pos / np.power(10000, 2.0 * (np.floor(i / 2.0)) / np.float32(D))
    angle[:, 0::2] = np.sin(angle[:, 0::2])
    angle[:, 1::2] = np.cos(angle[:, 1::2])
    return angle


_POS = _pos_encoding_np()


def _sc_body(idx0_hbm, idx1_hbm, num0_hbm, num1_hbm, ta_hbm, tr_hbm, pos_hbm,
             out_hbm,
             idx0a_v, idx0b_v, idx1a_v, idx1b_v, num0a_v, num0b_v,
             num1a_v, num1b_v, st0a_v, st0b_v, st1a_v, st1b_v, buf_v, pos_v,
             sem_ga, sem_gb, sem_o):
    wid = lax.axis_index("s") * NC + lax.axis_index("c")
    pltpu.sync_copy(pos_hbm, pos_v)

    idx0_slots = [idx0a_v, idx0b_v]
    idx1_slots = [idx1a_v, idx1b_v]
    num0_slots = [num0a_v, num0b_v]
    num1_slots = [num1a_v, num1b_v]
    st0_slots = [st0a_v, st0b_v]
    st1_slots = [st1a_v, st1b_v]
    g_sems = [sem_ga, sem_gb]
    b00 = wid * BPW

    def stage(k, p):
        b0 = b00 + k * NB
        pltpu.sync_copy(idx0_hbm.at[pl.ds(b0, NB)], idx0_slots[p])
        pltpu.sync_copy(idx1_hbm.at[pl.ds(b0, NB)], idx1_slots[p])
        pltpu.sync_copy(num0_hbm.at[pl.ds(b0, NB)], num0_slots[p])
        pltpu.sync_copy(num1_hbm.at[pl.ds(b0, NB)], num1_slots[p])
        for b in range(NB):
            pltpu.async_copy(ta_hbm.at[idx0_slots[p].at[b]],
                             st0_slots[p].at[pl.ds(b * GL, GL)], g_sems[p])
            pltpu.async_copy(tr_hbm.at[idx1_slots[p].at[b]],
                             st1_slots[p].at[pl.ds(b * GL, GL)], g_sems[p])

    def drain_gathers(p):
        pltpu.make_async_copy(
            ta_hbm.at[pl.ds(0, GL)],
            st0_slots[p].at[pl.ds(0, GL)], g_sems[p]).wait()
        pltpu.make_async_copy(
            tr_hbm.at[pl.ds(0, GL)],
            st1_slots[p].at[pl.ds(0, GL)], g_sems[p]).wait()
        pltpu.make_async_copy(
            ta_hbm.at[pl.ds(0, GL)],
            st0_slots[p].at[pl.ds(GL, GL)], g_sems[p]).wait()
        pltpu.make_async_copy(
            tr_hbm.at[pl.ds(0, GL)],
            st1_slots[p].at[pl.ds(GL, GL)], g_sems[p]).wait()

    def drain_out():
        pltpu.make_async_copy(
            buf_v, out_hbm.at[pl.ds(0, NB)], sem_o).wait()

    stage(0, 0)
    pltpu.async_copy(buf_v, out_hbm.at[pl.ds(b00, NB)], sem_o)

    def do_chunk(k, p):
        stage(lax.rem(k + 1, NCHUNK), 1 - p)
        drain_gathers(p)
        drain_out()

        idx0_v, idx1_v = idx0_slots[p], idx1_slots[p]
        num0_v, num1_v = num0_slots[p], num1_slots[p]
        st0, st1 = st0_slots[p], st1_slots[p]

        def ev_body(c, carry2):
            for b in range(NB):
                e = b * GL + c
                bsp = jnp.full((16,), b, jnp.int32)
                csp = jnp.full((16,), 0, jnp.int32) + c
                n0 = plsc.load_gather(num0_v, [bsp, csp])
                n1 = plsc.load_gather(num1_v, [bsp, csp])
                for j in range(D // 16):
                    pvec = pos_v[c, pl.ds(j * 16, 16)]
                    v0 = st0[e, pl.ds(j * 16, 16)]
                    v1 = st1[e, pl.ds(j * 16, 16)]
                    buf_v[b, c * A, pl.ds(j * 16, 16)] = v0 + pvec
                    buf_v[b, c * A + 1, pl.ds(j * 16, 16)] = v1 + pvec
                    buf_v[b, c * A + 2, pl.ds(j * 16, 16)] = n0 + pvec
                    buf_v[b, c * A + 3, pl.ds(j * 16, 16)] = n1 + pvec
            return carry2

        lax.fori_loop(0, C, ev_body, 0)

        pltpu.async_copy(buf_v, out_hbm.at[pl.ds(b00 + k * NB, NB)], sem_o)

    def chunk_pair(i, carry):
        do_chunk(2 * i, 0)
        do_chunk(2 * i + 1, 1)
        return carry

    lax.fori_loop(0, NCHUNK // 2, chunk_pair, 0)
    drain_out()
    drain_gathers(0)


def kernel(inputs, table_activity, table_resource):
    pos = jnp.asarray(_POS)
    idx0 = jnp.pad(inputs[:, 0::4].astype(jnp.int32), ((0, 0), (0, 6)))
    idx1 = jnp.pad(inputs[:, 1::4].astype(jnp.int32), ((0, 0), (0, 6)))
    num0 = inputs[:, 2::4]
    num1 = inputs[:, 3::4]
    ta128 = jnp.pad(table_activity, ((0, 0), (0, 128 - D)))
    tr128 = jnp.pad(table_resource, ((0, 0), (0, 128 - D)))
    mesh = plsc.VectorSubcoreMesh(core_axis_name="c", subcore_axis_name="s")
    k = functools.partial(
        pl.kernel,
        out_type=jax.ShapeDtypeStruct((B, CA, D), jnp.float32),
        mesh=mesh,
        compiler_params=pltpu.CompilerParams(use_tc_tiling_on_sc=True,
                                             needs_layout_passes=False),
        scratch_types=(
            [pltpu.VMEM((NB, GL), jnp.int32)] * 4 +
            [pltpu.VMEM((NB, C), jnp.float32)] * 4 +
            [pltpu.VMEM((EV, 128), jnp.float32)] * 4 +
            [pltpu.VMEM((NB, CA, D), jnp.float32),
             pltpu.VMEM((C, D), jnp.float32),
             pltpu.SemaphoreType.DMA,
             pltpu.SemaphoreType.DMA,
             pltpu.SemaphoreType.DMA]
        ),
    )(_sc_body)
    out = k(idx0, idx1, num0, num1, ta128, tr128, pos)
    return jex_layout.with_layout_constraint(
        out, jex_layout.Layout(major_to_minor=(0, 1, 2)))


# speedup vs baseline: 1.0608x; 1.0608x over previous
import functools

import jax
import jax.numpy as jnp
import numpy as np
from jax import lax
from jax.experimental import pallas as pl
from jax.experimental.pallas import tpu as pltpu
from jax.experimental.pallas import tpu_sc as plsc
from jax.experimental import layout as jex_layout

B, C, A, D = 1024, 50, 4, 64
VOCAB0, VOCAB1 = 100000, 1000
CA = C * A

NC, NS = 2, 16
NW = NC * NS
BPW = B // NW
NB = 2
NCHUNK = BPW // NB
EV = NB * C


def _pos_encoding_np():
    pos = np.arange(C)[:, np.newaxis].astype(np.float32)
    i = np.arange(D)[np.newaxis, :].astype(np.float32)
    angle = pos / np.power(10000, 2.0 * (np.floor(i / 2.0)) / np.float32(D))
    angle[:, 0::2] = np.sin(angle[:, 0::2])
    angle[:, 1::2] = np.cos(angle[:, 1::2])
    return angle


_POS = _pos_encoding_np()


def _sc_body(idx_hbm, num_hbm, ta_hbm, tr_hbm, pos_hbm,
             out_hbm, idx_v, num_v, st0_v, st1_v, buf_v,
             pos_v, sem):
    wid = lax.axis_index("s") * NC + lax.axis_index("c")
    pltpu.sync_copy(pos_hbm, pos_v)

    def chunk(k, carry):
        b0 = wid * BPW + k * NB
        pltpu.sync_copy(idx_hbm.at[pl.ds(b0, NB)], idx_v)
        pltpu.sync_copy(num_hbm.at[pl.ds(b0, NB)], num_v)

        cps = []
        for b in range(NB):
            cps.append(pltpu.async_copy(
                ta_hbm.at[idx_v.at[b, pl.ds(0, 56)]],
                st0_v.at[pl.ds(b * 56, 56)], sem))
            cps.append(pltpu.async_copy(
                tr_hbm.at[idx_v.at[b, pl.ds(56, 56)]],
                st1_v.at[pl.ds(b * 56, 56)], sem))
        for cp in cps:
            cp.wait()

        def ev_body(c, carry2):
            pv = [pos_v[c, pl.ds(j * 16, 16)] for j in range(D // 16)]
            for b in range(NB):
                e = b * 56 + c
                bsp = jnp.full((16,), b, jnp.int32)
                csp = jnp.full((16,), 0, jnp.int32) + c
                n0 = plsc.load_gather(num_v, [bsp, csp])
                n1 = plsc.load_gather(num_v, [bsp, csp + 56])
                for j in range(D // 16):
                    p = pv[j]
                    v0 = st0_v[e, pl.ds(j * 16, 16)]
                    v1 = st1_v[e, pl.ds(j * 16, 16)]
                    buf_v[b, c * A, pl.ds(j * 16, 16)] = v0 + p
                    buf_v[b, c * A + 1, pl.ds(j * 16, 16)] = v1 + p
                    buf_v[b, c * A + 2, pl.ds(j * 16, 16)] = n0 + p
                    buf_v[b, c * A + 3, pl.ds(j * 16, 16)] = n1 + p
            return carry2

        lax.fori_loop(0, C, ev_body, 0)

        pltpu.sync_copy(buf_v, out_hbm.at[pl.ds(b0, NB)])
        return carry

    lax.fori_loop(0, NCHUNK, chunk, 0)


def kernel(inputs, table_activity, table_resource):
    pos = jnp.asarray(_POS)
    idx0 = inputs[:, 0::4].astype(jnp.int32)
    idx1 = inputs[:, 1::4].astype(jnp.int32)
    z6i = jnp.zeros((B, 6), jnp.int32)
    idx = jnp.concatenate([idx0, z6i, idx1, z6i], axis=1)
    z6f = jnp.zeros((B, 6), jnp.float32)
    num = jnp.concatenate([inputs[:, 2::4], z6f, inputs[:, 3::4], z6f],
                          axis=1)
    ta128 = jnp.pad(table_activity, ((0, 0), (0, 128 - D)))
    tr128 = jnp.pad(table_resource, ((0, 0), (0, 128 - D)))
    mesh = plsc.VectorSubcoreMesh(core_axis_name="c", subcore_axis_name="s")
    k = functools.partial(
        pl.kernel,
        out_type=jax.ShapeDtypeStruct((B, CA, D), jnp.float32),
        mesh=mesh,
        compiler_params=pltpu.CompilerParams(use_tc_tiling_on_sc=True,
                                             needs_layout_passes=False),
        scratch_types=[
            pltpu.VMEM((NB, 112), jnp.int32),
            pltpu.VMEM((NB, 112), jnp.float32),
            pltpu.VMEM((NB * 56, 128), jnp.float32),
            pltpu.VMEM((NB * 56, 128), jnp.float32),
            pltpu.VMEM((NB, CA, D), jnp.float32),
            pltpu.VMEM((C, D), jnp.float32),
            pltpu.SemaphoreType.DMA,
        ],
    )(_sc_body)
    out = k(idx, num, ta128, tr128, pos)
    return jex_layout.with_layout_constraint(
        out, jex_layout.Layout(major_to_minor=(0, 1, 2)))
